# baseline (speedup 1.0000x reference)
import functools

import jax
import jax.numpy as jnp
from jax import lax
from jax.experimental import pallas as pl
from jax.experimental.pallas import tpu as pltpu
from jax.experimental.pallas import tpu_sc as plsc

B, H, W = 8, 512, 512
N = B * H * W
RB = 64
NB = 2048
NC, NS = 2, 16
NW = NC * NS
PER_W = N // NW
CH = 8192
EPS = 1e-30
EPS2 = 1e-35

S_NPOS, S_LPOS, S_NAVAIL, S_NEGSUM, S_NEGMIN, S_NEGMAX = 0, 1, 2, 3, 4, 5
S_TCLSUM, S_TCLTRN, S_MCNT, S_RADSUM, S_SINSUM, S_COSSUM = 6, 7, 8, 9, 10, 11
P_LO1, P_RNG1, P_B1, P_LO2, P_W1, P_CAB = 0, 1, 2, 3, 4, 5
T_TLO, T_TAU = 0, 1


def _ce2(a0, a1, tgt):
    m = jnp.maximum(a0, a1)
    lse = m + jnp.log1p(jnp.exp(-jnp.abs(a0 - a1)))
    picked = jnp.where(tgt > 0, a1, a0)
    return lse - picked


def _smooth_l1(d):
    ad = jnp.abs(d)
    return jnp.where(ad < 1.0, 0.5 * d * d, ad - 0.5)


def _pass_a_body(inp_ref, tr_ref, tcl_ref, sin_ref, cos_ref, rad_ref, trn_ref,
                 sc_ref, neg_ref):
    b = pl.program_id(0)
    h = pl.program_id(1)
    first = jnp.logical_and(b == 0, h == 0)

    @pl.when(first)
    def _init():
        for i in range(16):
            sc_ref[i] = jnp.float32(0.0)
        sc_ref[S_NEGMIN] = jnp.float32(jnp.inf)
        sc_ref[S_NEGMAX] = jnp.float32(-jnp.inf)

    x = inp_ref[0]
    trm = tr_ref[0]
    tclm = tcl_ref[0]
    trnm = trn_ref[0]
    sinm = sin_ref[0]
    cosm = cos_ref[0]
    radm = rad_ref[0]

    f32 = jnp.float32
    train = trnm > 0
    pos = jnp.logical_and(trm > 0, train)
    neg = jnp.logical_and(trm == 0, train)

    l_tr = _ce2(x[0], x[1], trm)
    l_tcl = _ce2(x[2], x[3], tclm)

    neg_v = jnp.where(neg, l_tr, f32(-1.0))
    neg_ref[...] = neg_v

    sc_ref[S_NPOS] = sc_ref[S_NPOS] + jnp.sum(pos.astype(f32))
    sc_ref[S_LPOS] = sc_ref[S_LPOS] + jnp.sum(jnp.where(pos, l_tr, 0.0))
    sc_ref[S_NAVAIL] = sc_ref[S_NAVAIL] + jnp.sum(neg.astype(f32))
    sc_ref[S_NEGSUM] = sc_ref[S_NEGSUM] + jnp.sum(jnp.where(neg, l_tr, 0.0))
    sc_ref[S_NEGMIN] = jnp.minimum(
        sc_ref[S_NEGMIN], jnp.min(jnp.where(neg, l_tr, f32(jnp.inf))))
    sc_ref[S_NEGMAX] = jnp.maximum(
        sc_ref[S_NEGMAX], jnp.max(jnp.where(neg, l_tr, f32(-jnp.inf))))

    sc_ref[S_TCLSUM] = sc_ref[S_TCLSUM] + jnp.sum(jnp.where(pos, l_tcl, 0.0))
    sc_ref[S_TCLTRN] = sc_ref[S_TCLTRN] + jnp.sum(
        jnp.logical_and(tclm > 0, train).astype(f32))

    m = tclm > 0
    sc_ref[S_MCNT] = sc_ref[S_MCNT] + jnp.sum(m.astype(f32))

    scale = jnp.sqrt(1.0 / (x[4] * x[4] + x[5] * x[5]))
    sin_p = x[4] * scale
    cos_p = x[5] * scale
    sc_ref[S_RADSUM] = sc_ref[S_RADSUM] + jnp.sum(
        jnp.where(m, _smooth_l1(x[6] / radm - 1.0), 0.0))
    sc_ref[S_SINSUM] = sc_ref[S_SINSUM] + jnp.sum(
        jnp.where(m, _smooth_l1(sin_p - sinm), 0.0))
    sc_ref[S_COSSUM] = sc_ref[S_COSSUM] + jnp.sum(
        jnp.where(m, _smooth_l1(cos_p - cosm), 0.0))


def _pass_a(input, tr_mask, tcl_mask, sin_map, cos_map, radii_map, train_mask):
    hb = H // RB
    mask_spec = pl.BlockSpec((1, RB, W), lambda b, h: (b, h, 0))
    return pl.pallas_call(
        _pass_a_body,
        grid=(B, hb),
        in_specs=[
            pl.BlockSpec((1, 7, RB, W), lambda b, h: (b, 0, h, 0)),
            mask_spec, mask_spec, mask_spec, mask_spec, mask_spec, mask_spec,
        ],
        out_specs=[
            pl.BlockSpec(memory_space=pltpu.SMEM),
            pl.BlockSpec((RB, W), lambda b, h: (b * hb + h, 0)),
        ],
        out_shape=[
            jax.ShapeDtypeStruct((16,), jnp.float32),
            jax.ShapeDtypeStruct((B * H, W), jnp.float32),
        ],
    )(input, tr_mask, tcl_mask, sin_map, cos_map, radii_map, train_mask)


def _lane(vec_ref, i):
    v = vec_ref[...]
    idx = jnp.full((16, 1), i, jnp.int32)
    dn = lax.GatherDimensionNumbers(
        offset_dims=(), collapsed_slice_dims=(0,), start_index_map=(0,))
    return lax.gather(v, idx, dn, (1,), mode=lax.GatherScatterMode.PROMISE_IN_BOUNDS)


def _sc_hist_body(round2, vals, params, cnt_out, pbuf, vbuf, hcnt):
    wid = lax.axis_index("s") * NC + lax.axis_index("c")
    pltpu.sync_copy(params, pbuf)

    zeros = jnp.zeros((16,), jnp.float32)
    ones = jnp.ones((16,), jnp.float32)

    def _zero(i, _):
        hcnt[pl.ds(i * 16, 16)] = zeros
        return 0
    lax.fori_loop(0, NB // 16, _zero, 0)

    nbf = jnp.float32(NB)
    if not round2:
        lo = _lane(pbuf, S_NEGMIN)
        rng = jnp.maximum(_lane(pbuf, S_NEGMAX) - lo, EPS)
        scale = nbf / rng
    else:
        lo1 = _lane(pbuf, P_LO1)
        scale1 = nbf / _lane(pbuf, P_RNG1)
        b1 = _lane(pbuf, P_B1).astype(jnp.int32)
        lo = _lane(pbuf, P_LO2)
        scale = nbf / jnp.maximum(_lane(pbuf, P_W1), EPS2)

    def _elem(j, _):
        v = vbuf[pl.ds(j * 16, 16)]
        valid = v >= 0.0
        if round2:
            idx1 = jnp.clip(((v - lo1) * scale1).astype(jnp.int32), 0, NB - 1)
            valid = jnp.logical_and(valid, idx1 == b1)
        idx = jnp.clip(((v - lo) * scale).astype(jnp.int32), 0, NB - 1)
        plsc.addupdate_scatter(hcnt, [idx], ones, mask=valid)
        return 0

    base = wid * PER_W
    for c in range(PER_W // CH):
        pltpu.sync_copy(vals.at[pl.ds(base + c * CH, CH)], vbuf)
        lax.fori_loop(0, CH // 16, _elem, 0)

    pltpu.sync_copy(hcnt, cnt_out.at[wid])


def _sc_hist(vals, params, round2):
    body = functools.partial(_sc_hist_body, round2)
    return pl.kernel(
        body,
        out_type=jax.ShapeDtypeStruct((NW, NB), jnp.float32),
        mesh=plsc.VectorSubcoreMesh(core_axis_name="c", subcore_axis_name="s",
                                    num_cores=NC, num_subcores=NS),
        compiler_params=pltpu.CompilerParams(needs_layout_passes=False),
        scratch_types=[
            pltpu.VMEM((16,), jnp.float32),
            pltpu.VMEM((CH,), jnp.float32),
            pltpu.VMEM((NB,), jnp.float32),
        ],
    )(vals, params)


def _suffix_counts(cnt):
    v = cnt.reshape(16, 128)
    r = jnp.arange(16, dtype=jnp.int32)
    row_after = (r[None, :] > r[:, None]).astype(jnp.float32)
    c = jnp.arange(128, dtype=jnp.int32)
    within = (c[None, :] >= c[:, None]).astype(jnp.float32)
    ccnt = jnp.dot(v, within.T, preferred_element_type=jnp.float32) + \
        jnp.dot(row_after, jnp.sum(v, axis=1))[:, None]
    return ccnt.reshape(NB)


def _pick_bin(ccnt, k):
    bf = jnp.sum((ccnt >= k).astype(jnp.float32)) - 1.0
    j = jnp.arange(NB, dtype=jnp.int32).astype(jnp.float32)
    c_ab = jnp.sum(jnp.where(j == (bf + 1.0), ccnt, 0.0))
    return bf, c_ab


def _ohem_k(sc):
    npos = sc[S_NPOS]
    avail = sc[S_NAVAIL]
    return jnp.where(npos > 0, jnp.minimum(avail, 3.0 * npos),
                     jnp.float32(100.0))


def _c1_body(cnt_ref, sc_ref, p_ref):
    ccnt = _suffix_counts(jnp.sum(cnt_ref[...], axis=0))
    sc = [sc_ref[i] for i in range(12)]
    k = _ohem_k(sc)
    bf, c_ab = _pick_bin(ccnt, k)
    lo1 = sc[S_NEGMIN]
    rng1 = jnp.maximum(sc[S_NEGMAX] - lo1, EPS)
    w1 = rng1 / jnp.float32(NB)
    for i in range(8):
        p_ref[i] = jnp.float32(0.0)
    p_ref[P_LO1] = lo1
    p_ref[P_RNG1] = rng1
    p_ref[P_B1] = bf
    p_ref[P_LO2] = lo1 + bf * w1
    p_ref[P_W1] = w1
    p_ref[P_CAB] = c_ab


def _c1(cnt1, scalars):
    return pl.pallas_call(
        _c1_body,
        in_specs=[
            pl.BlockSpec(memory_space=pltpu.VMEM),
            pl.BlockSpec(memory_space=pltpu.SMEM),
        ],
        out_specs=pl.BlockSpec(memory_space=pltpu.SMEM),
        out_shape=jax.ShapeDtypeStruct((16,), jnp.float32),
    )(cnt1, scalars)


def _c2_body(cnt_ref, p_ref, sc_ref, t_ref):
    ccnt = _suffix_counts(jnp.sum(cnt_ref[...], axis=0))
    sc = [sc_ref[i] for i in range(12)]
    k = _ohem_k(sc)
    b2, _ = _pick_bin(ccnt, k - p_ref[P_CAB])
    w2 = p_ref[P_W1] / jnp.float32(NB)
    t_lo = p_ref[P_LO2] + b2 * w2
    tau = t_lo + 0.5 * w2
    for i in range(8):
        t_ref[i] = jnp.float32(0.0)
    t_ref[T_TLO] = t_lo
    t_ref[T_TAU] = tau


def _c2(cnt2, params2, scalars):
    return pl.pallas_call(
        _c2_body,
        in_specs=[
            pl.BlockSpec(memory_space=pltpu.VMEM),
            pl.BlockSpec(memory_space=pltpu.SMEM),
            pl.BlockSpec(memory_space=pltpu.SMEM),
        ],
        out_specs=pl.BlockSpec(memory_space=pltpu.SMEM),
        out_shape=jax.ShapeDtypeStruct((8,), jnp.float32),
    )(cnt2, params2, scalars)


def _d_body(neg_ref, t_ref, sc_ref, out_ref, acc_ref):
    i = pl.program_id(0)

    @pl.when(i == 0)
    def _init():
        acc_ref[0] = jnp.float32(0.0)
        acc_ref[1] = jnp.float32(0.0)

    v = neg_ref[...]
    t = t_ref[T_TLO]
    gt = v > t
    acc_ref[0] = acc_ref[0] + jnp.sum(gt.astype(jnp.float32))
    acc_ref[1] = acc_ref[1] + jnp.sum(jnp.where(gt, v, 0.0))

    @pl.when(i == pl.num_programs(0) - 1)
    def _fin():
        sc = [sc_ref[j] for j in range(12)]
        k = _ohem_k(sc)
        cnt_gt = acc_ref[0]
        sum_gt = acc_ref[1]
        deficit = k - cnt_gt
        top = sum_gt + deficit * t_ref[T_TAU]
        avail = sc[S_NAVAIL]
        top = jnp.where(k <= 0.0, 0.0,
                        jnp.where(k > avail, jnp.float32(-jnp.inf), top))
        npos = sc[S_NPOS]
        loss_tr = (sc[S_LPOS] + top) / (npos + k)
        loss_tcl = jnp.where(
            npos > 0, sc[S_TCLSUM] / jnp.maximum(npos, 1.0), jnp.float32(0.0))
        gate = sc[S_TCLTRN] > 0
        mc = jnp.maximum(sc[S_MCNT], 1.0)
        out_ref[0] = loss_tr
        out_ref[1] = loss_tcl
        out_ref[2] = jnp.where(gate, sc[S_RADSUM] / mc, jnp.float32(0.0))
        out_ref[3] = jnp.where(gate, sc[S_SINSUM] / mc, jnp.float32(0.0))
        out_ref[4] = jnp.where(gate, sc[S_COSSUM] / mc, jnp.float32(0.0))
        out_ref[5] = jnp.float32(0.0)
        out_ref[6] = jnp.float32(0.0)
        out_ref[7] = jnp.float32(0.0)


def _d(negv, tparams, scalars):
    nblk = (B * H) // RB
    return pl.pallas_call(
        _d_body,
        grid=(nblk,),
        in_specs=[
            pl.BlockSpec((RB, W), lambda i: (i, 0)),
            pl.BlockSpec(memory_space=pltpu.SMEM),
            pl.BlockSpec(memory_space=pltpu.SMEM),
        ],
        out_specs=pl.BlockSpec(memory_space=pltpu.SMEM),
        out_shape=jax.ShapeDtypeStruct((8,), jnp.float32),
        scratch_shapes=[pltpu.SMEM((2,), jnp.float32)],
    )(negv, tparams, scalars)


def kernel(input, tr_mask, tcl_mask, sin_map, cos_map, radii_map, train_mask):
    scalars, negv = _pass_a(
        input, tr_mask, tcl_mask, sin_map, cos_map, radii_map, train_mask)
    negf = negv.reshape(-1)
    cnt1 = _sc_hist(negf, scalars, round2=False)
    p2 = _c1(cnt1, scalars)
    cnt2 = _sc_hist(negf, p2, round2=True)
    tp = _c2(cnt2, p2, scalars)
    out = _d(negv, tp, scalars)
    return (out[0], out[1], out[2], out[3], out[4])


# speedup vs baseline: 1.7792x; 1.7792x over previous
import jax
import jax.numpy as jnp
from jax import lax
from jax.experimental import pallas as pl
from jax.experimental.pallas import tpu as pltpu
from jax.experimental.pallas import tpu_sc as plsc

B, H, W = 8, 512, 512
N = B * H * W
RB = 64
NB = 32768
SHIFT = 16
NC, NS = 2, 16
NW = NC * NS
PER_W = N // NW
CH = 8192
UNROLL = 8

S_NPOS, S_LPOS, S_NAVAIL, S_TCLSUM, S_TCLTRN = 0, 1, 2, 3, 4
S_MCNT, S_RADSUM, S_SINSUM, S_COSSUM = 5, 6, 7, 8


def _ce2(a0, a1, tgt):
    m = jnp.maximum(a0, a1)
    lse = m + jnp.log1p(jnp.exp(-jnp.abs(a0 - a1)))
    picked = jnp.where(tgt > 0, a1, a0)
    return lse - picked


def _smooth_l1(d):
    ad = jnp.abs(d)
    return jnp.where(ad < 1.0, 0.5 * d * d, ad - 0.5)


def _pass_a_body(inp_ref, tr_ref, tcl_ref, sin_ref, cos_ref, rad_ref, trn_ref,
                 sc_ref, neg_ref):
    b = pl.program_id(0)
    h = pl.program_id(1)

    @pl.when(jnp.logical_and(b == 0, h == 0))
    def _init():
        for i in range(16):
            sc_ref[i] = jnp.float32(0.0)

    x = inp_ref[0]
    trm = tr_ref[0]
    tclm = tcl_ref[0]
    trnm = trn_ref[0]
    sinm = sin_ref[0]
    cosm = cos_ref[0]
    radm = rad_ref[0]

    f32 = jnp.float32
    train = trnm > 0
    pos = jnp.logical_and(trm > 0, train)
    neg = jnp.logical_and(trm == 0, train)

    l_tr = _ce2(x[0], x[1], trm)
    l_tcl = _ce2(x[2], x[3], tclm)

    neg_ref[...] = jnp.where(neg, l_tr, f32(-1.0))

    sc_ref[S_NPOS] = sc_ref[S_NPOS] + jnp.sum(pos.astype(f32))
    sc_ref[S_LPOS] = sc_ref[S_LPOS] + jnp.sum(jnp.where(pos, l_tr, 0.0))
    sc_ref[S_NAVAIL] = sc_ref[S_NAVAIL] + jnp.sum(neg.astype(f32))

    sc_ref[S_TCLSUM] = sc_ref[S_TCLSUM] + jnp.sum(jnp.where(pos, l_tcl, 0.0))
    sc_ref[S_TCLTRN] = sc_ref[S_TCLTRN] + jnp.sum(
        jnp.logical_and(tclm > 0, train).astype(f32))

    m = tclm > 0
    sc_ref[S_MCNT] = sc_ref[S_MCNT] + jnp.sum(m.astype(f32))

    scale = jnp.sqrt(1.0 / (x[4] * x[4] + x[5] * x[5]))
    sc_ref[S_RADSUM] = sc_ref[S_RADSUM] + jnp.sum(
        jnp.where(m, _smooth_l1(x[6] / radm - 1.0), 0.0))
    sc_ref[S_SINSUM] = sc_ref[S_SINSUM] + jnp.sum(
        jnp.where(m, _smooth_l1(x[4] * scale - sinm), 0.0))
    sc_ref[S_COSSUM] = sc_ref[S_COSSUM] + jnp.sum(
        jnp.where(m, _smooth_l1(x[5] * scale - cosm), 0.0))


def _pass_a(input, tr_mask, tcl_mask, sin_map, cos_map, radii_map, train_mask):
    hb = H // RB
    mask_spec = pl.BlockSpec((1, RB, W), lambda b, h: (b, h, 0))
    return pl.pallas_call(
        _pass_a_body,
        grid=(B, hb),
        in_specs=[
            pl.BlockSpec((1, 7, RB, W), lambda b, h: (b, 0, h, 0)),
            mask_spec, mask_spec, mask_spec, mask_spec, mask_spec, mask_spec,
        ],
        out_specs=[
            pl.BlockSpec(memory_space=pltpu.SMEM),
            pl.BlockSpec((RB, W), lambda b, h: (b * hb + h, 0)),
        ],
        out_shape=[
            jax.ShapeDtypeStruct((16,), jnp.float32),
            jax.ShapeDtypeStruct((B * H, W), jnp.float32),
        ],
    )(input, tr_mask, tcl_mask, sin_map, cos_map, radii_map, train_mask)


def _sc_hist_body(vals, cnt_out, sum_out, vbuf, hcnt, hsum):
    wid = lax.axis_index("s") * NC + lax.axis_index("c")

    zeros = jnp.zeros((16,), jnp.float32)
    ones = jnp.ones((16,), jnp.float32)

    def _zero(i, _):
        hcnt[pl.ds(i * 16, 16)] = zeros
        hsum[pl.ds(i * 16, 16)] = zeros
        return 0
    lax.fori_loop(0, NB // 16, _zero, 0)

    def _elem(j, _):
        for u in range(UNROLL):
            v = vbuf[pl.ds((j * UNROLL + u) * 16, 16)]
            valid = v >= 0.0
            idx = lax.shift_right_arithmetic(
                lax.bitcast_convert_type(v, jnp.int32), SHIFT)
            plsc.addupdate_scatter(hcnt, [idx], ones, mask=valid)
            plsc.addupdate_scatter(hsum, [idx], v, mask=valid)
        return 0

    base = wid * PER_W
    for c in range(PER_W // CH):
        pltpu.sync_copy(vals.at[pl.ds(base + c * CH, CH)], vbuf)
        lax.fori_loop(0, CH // (16 * UNROLL), _elem, 0)

    pltpu.sync_copy(hcnt, cnt_out.at[wid])
    pltpu.sync_copy(hsum, sum_out.at[wid])


def _sc_hist(vals):
    return pl.kernel(
        _sc_hist_body,
        out_type=(
            jax.ShapeDtypeStruct((NW, NB), jnp.float32),
            jax.ShapeDtypeStruct((NW, NB), jnp.float32),
        ),
        mesh=plsc.VectorSubcoreMesh(core_axis_name="c", subcore_axis_name="s",
                                    num_cores=NC, num_subcores=NS),
        compiler_params=pltpu.CompilerParams(needs_layout_passes=False),
        scratch_types=[
            pltpu.VMEM((CH,), jnp.float32),
            pltpu.VMEM((NB,), jnp.float32),
            pltpu.VMEM((NB,), jnp.float32),
        ],
    )(vals)


def _suffix(table):
    rows = NB // 128
    v = table.reshape(rows, 128)
    r = jnp.arange(rows, dtype=jnp.int32)
    row_after = (r[None, :] > r[:, None]).astype(jnp.float32)
    c = jnp.arange(128, dtype=jnp.int32)
    within = (c[None, :] >= c[:, None]).astype(jnp.float32)
    out = jnp.dot(v, within.T, preferred_element_type=jnp.float32) + \
        jnp.dot(row_after, jnp.sum(v, axis=1))[:, None]
    return out.reshape(NB)


def _final_body(cnt_ref, sum_ref, sc_ref, out_ref):
    f32 = jnp.float32
    cnt = jnp.sum(cnt_ref[...], axis=0)
    sums = jnp.sum(sum_ref[...], axis=0)
    ccnt = _suffix(cnt)
    csum = _suffix(sums)

    sc = [sc_ref[i] for i in range(9)]
    npos = sc[S_NPOS]
    avail = sc[S_NAVAIL]
    k = jnp.where(npos > 0, jnp.minimum(avail, 3.0 * npos), f32(100.0))

    bf = jnp.sum((ccnt >= k).astype(f32)) - 1.0
    j = jnp.arange(NB, dtype=jnp.int32).astype(f32)
    above = j == (bf + 1.0)
    c_ab = jnp.sum(jnp.where(above, ccnt, 0.0))
    s_ab = jnp.sum(jnp.where(above, csum, 0.0))

    b_i = bf.astype(jnp.int32)
    v_lo = lax.bitcast_convert_type(
        lax.shift_left(b_i, SHIFT), jnp.float32)
    v_hi = lax.bitcast_convert_type(
        lax.shift_left(b_i + 1, SHIFT), jnp.float32)
    tau = 0.5 * (v_lo + v_hi)

    deficit = k - c_ab
    top = s_ab + deficit * tau
    top = jnp.where(k <= 0.0, 0.0,
                    jnp.where(k > avail, f32(-jnp.inf), top))

    loss_tr = (sc[S_LPOS] + top) / (npos + k)
    loss_tcl = jnp.where(
        npos > 0, sc[S_TCLSUM] / jnp.maximum(npos, 1.0), f32(0.0))
    gate = sc[S_TCLTRN] > 0
    mc = jnp.maximum(sc[S_MCNT], 1.0)
    out_ref[0] = loss_tr
    out_ref[1] = loss_tcl
    out_ref[2] = jnp.where(gate, sc[S_RADSUM] / mc, f32(0.0))
    out_ref[3] = jnp.where(gate, sc[S_SINSUM] / mc, f32(0.0))
    out_ref[4] = jnp.where(gate, sc[S_COSSUM] / mc, f32(0.0))
    out_ref[5] = f32(0.0)
    out_ref[6] = f32(0.0)
    out_ref[7] = f32(0.0)


def _final(cnt, sums, scalars):
    return pl.pallas_call(
        _final_body,
        in_specs=[
            pl.BlockSpec(memory_space=pltpu.VMEM),
            pl.BlockSpec(memory_space=pltpu.VMEM),
            pl.BlockSpec(memory_space=pltpu.SMEM),
        ],
        out_specs=pl.BlockSpec(memory_space=pltpu.SMEM),
        out_shape=jax.ShapeDtypeStruct((8,), jnp.float32),
    )(cnt, sums, scalars)


def kernel(input, tr_mask, tcl_mask, sin_map, cos_map, radii_map, train_mask):
    scalars, negv = _pass_a(
        input, tr_mask, tcl_mask, sin_map, cos_map, radii_map, train_mask)
    cnt, sums = _sc_hist(negv.reshape(-1))
    out = _final(cnt, sums, scalars)
    return (out[0], out[1], out[2], out[3], out[4])


# speedup vs baseline: 2.1831x; 1.2270x over previous
import jax
import jax.numpy as jnp
from jax import lax
from jax.experimental import pallas as pl
from jax.experimental.pallas import tpu as pltpu
from jax.experimental.pallas import tpu_sc as plsc

B, H, W = 8, 512, 512
N = B * H * W
RB = 64
NB = 32768
SHIFT = 16
NC, NS = 2, 16
NW = NC * NS
PER_W = N // NW
CH = 8192
UNROLL = 8

S_NPOS, S_LPOS, S_NAVAIL, S_TCLSUM, S_TCLTRN = 0, 1, 2, 3, 4
S_MCNT, S_RADSUM, S_SINSUM, S_COSSUM = 5, 6, 7, 8


def _ce2(a0, a1, tgt):
    m = jnp.maximum(a0, a1)
    lse = m + jnp.log1p(jnp.exp(-jnp.abs(a0 - a1)))
    picked = jnp.where(tgt > 0, a1, a0)
    return lse - picked


def _smooth_l1(d):
    ad = jnp.abs(d)
    return jnp.where(ad < 1.0, 0.5 * d * d, ad - 0.5)


def _pass_a_body(inp_ref, tr_ref, tcl_ref, sin_ref, cos_ref, rad_ref, trn_ref,
                 sc_ref, neg_ref):
    b = pl.program_id(0)
    h = pl.program_id(1)

    @pl.when(jnp.logical_and(b == 0, h == 0))
    def _init():
        for i in range(16):
            sc_ref[i] = jnp.float32(0.0)

    x = inp_ref[0]
    trm = tr_ref[0]
    tclm = tcl_ref[0]
    trnm = trn_ref[0]
    sinm = sin_ref[0]
    cosm = cos_ref[0]
    radm = rad_ref[0]

    f32 = jnp.float32
    train = trnm > 0
    pos = jnp.logical_and(trm > 0, train)
    neg = jnp.logical_and(trm == 0, train)

    l_tr = _ce2(x[0], x[1], trm)
    l_tcl = _ce2(x[2], x[3], tclm)

    neg_ref[...] = jnp.where(neg, l_tr, f32(-1.0))

    sc_ref[S_NPOS] = sc_ref[S_NPOS] + jnp.sum(pos.astype(f32))
    sc_ref[S_LPOS] = sc_ref[S_LPOS] + jnp.sum(jnp.where(pos, l_tr, 0.0))
    sc_ref[S_NAVAIL] = sc_ref[S_NAVAIL] + jnp.sum(neg.astype(f32))

    sc_ref[S_TCLSUM] = sc_ref[S_TCLSUM] + jnp.sum(jnp.where(pos, l_tcl, 0.0))
    sc_ref[S_TCLTRN] = sc_ref[S_TCLTRN] + jnp.sum(
        jnp.logical_and(tclm > 0, train).astype(f32))

    m = tclm > 0
    sc_ref[S_MCNT] = sc_ref[S_MCNT] + jnp.sum(m.astype(f32))

    scale = jnp.sqrt(1.0 / (x[4] * x[4] + x[5] * x[5]))
    sc_ref[S_RADSUM] = sc_ref[S_RADSUM] + jnp.sum(
        jnp.where(m, _smooth_l1(x[6] / radm - 1.0), 0.0))
    sc_ref[S_SINSUM] = sc_ref[S_SINSUM] + jnp.sum(
        jnp.where(m, _smooth_l1(x[4] * scale - sinm), 0.0))
    sc_ref[S_COSSUM] = sc_ref[S_COSSUM] + jnp.sum(
        jnp.where(m, _smooth_l1(x[5] * scale - cosm), 0.0))


def _pass_a(input, tr_mask, tcl_mask, sin_map, cos_map, radii_map, train_mask):
    hb = H // RB
    mask_spec = pl.BlockSpec((1, RB, W), lambda b, h: (b, h, 0))
    return pl.pallas_call(
        _pass_a_body,
        grid=(B, hb),
        in_specs=[
            pl.BlockSpec((1, 7, RB, W), lambda b, h: (b, 0, h, 0)),
            mask_spec, mask_spec, mask_spec, mask_spec, mask_spec, mask_spec,
        ],
        out_specs=[
            pl.BlockSpec(memory_space=pltpu.SMEM),
            pl.BlockSpec((RB, W), lambda b, h: (b * hb + h, 0)),
        ],
        out_shape=[
            jax.ShapeDtypeStruct((16,), jnp.float32),
            jax.ShapeDtypeStruct((B * H, W), jnp.float32),
        ],
    )(input, tr_mask, tcl_mask, sin_map, cos_map, radii_map, train_mask)


def _sc_hist_body(vals, cnt_out, sum_out, vbuf, hcnt, hsum):
    wid = lax.axis_index("s") * NC + lax.axis_index("c")

    zeros = jnp.zeros((16,), jnp.float32)
    ones = jnp.ones((16,), jnp.float32)

    @plsc.parallel_loop(0, NB // 16, unroll=8)
    def _zero(i):
        hcnt[pl.ds(i * 16, 16)] = zeros
        hsum[pl.ds(i * 16, 16)] = zeros

    base = wid * PER_W
    for c in range(PER_W // CH):
        pltpu.sync_copy(vals.at[pl.ds(base + c * CH, CH)], vbuf)

        @plsc.parallel_loop(0, CH // 16, unroll=UNROLL)
        def _elem(j):
            v = vbuf[pl.ds(j * 16, 16)]
            valid = v >= 0.0
            idx = lax.shift_right_arithmetic(
                lax.bitcast_convert_type(v, jnp.int32), SHIFT)
            plsc.addupdate_scatter(hcnt, [idx], ones, mask=valid)
            plsc.addupdate_scatter(hsum, [idx], v, mask=valid)

    pltpu.sync_copy(hcnt, cnt_out.at[wid])
    pltpu.sync_copy(hsum, sum_out.at[wid])


def _sc_hist(vals):
    return pl.kernel(
        _sc_hist_body,
        out_type=(
            jax.ShapeDtypeStruct((NW, NB), jnp.float32),
            jax.ShapeDtypeStruct((NW, NB), jnp.float32),
        ),
        mesh=plsc.VectorSubcoreMesh(core_axis_name="c", subcore_axis_name="s",
                                    num_cores=NC, num_subcores=NS),
        compiler_params=pltpu.CompilerParams(needs_layout_passes=False),
        scratch_types=[
            pltpu.VMEM((CH,), jnp.float32),
            pltpu.VMEM((NB,), jnp.float32),
            pltpu.VMEM((NB,), jnp.float32),
        ],
    )(vals)


def _suffix(table):
    rows = NB // 128
    v = table.reshape(rows, 128)
    r = jnp.arange(rows, dtype=jnp.int32)
    row_after = (r[None, :] > r[:, None]).astype(jnp.float32)
    c = jnp.arange(128, dtype=jnp.int32)
    within = (c[None, :] >= c[:, None]).astype(jnp.float32)
    out = jnp.dot(v, within.T, preferred_element_type=jnp.float32) + \
        jnp.dot(row_after, jnp.sum(v, axis=1))[:, None]
    return out.reshape(NB)


def _final_body(cnt_ref, sum_ref, sc_ref, out_ref):
    f32 = jnp.float32
    cnt = jnp.sum(cnt_ref[...], axis=0)
    sums = jnp.sum(sum_ref[...], axis=0)
    ccnt = _suffix(cnt)
    csum = _suffix(sums)

    sc = [sc_ref[i] for i in range(9)]
    npos = sc[S_NPOS]
    avail = sc[S_NAVAIL]
    k = jnp.where(npos > 0, jnp.minimum(avail, 3.0 * npos), f32(100.0))

    bf = jnp.sum((ccnt >= k).astype(f32)) - 1.0
    j = jnp.arange(NB, dtype=jnp.int32).astype(f32)
    above = j == (bf + 1.0)
    c_ab = jnp.sum(jnp.where(above, ccnt, 0.0))
    s_ab = jnp.sum(jnp.where(above, csum, 0.0))

    b_i = bf.astype(jnp.int32)
    v_lo = lax.bitcast_convert_type(
        lax.shift_left(b_i, SHIFT), jnp.float32)
    v_hi = lax.bitcast_convert_type(
        lax.shift_left(b_i + 1, SHIFT), jnp.float32)
    tau = 0.5 * (v_lo + v_hi)

    deficit = k - c_ab
    top = s_ab + deficit * tau
    top = jnp.where(k <= 0.0, 0.0,
                    jnp.where(k > avail, f32(-jnp.inf), top))

    loss_tr = (sc[S_LPOS] + top) / (npos + k)
    loss_tcl = jnp.where(
        npos > 0, sc[S_TCLSUM] / jnp.maximum(npos, 1.0), f32(0.0))
    gate = sc[S_TCLTRN] > 0
    mc = jnp.maximum(sc[S_MCNT], 1.0)
    out_ref[0] = loss_tr
    out_ref[1] = loss_tcl
    out_ref[2] = jnp.where(gate, sc[S_RADSUM] / mc, f32(0.0))
    out_ref[3] = jnp.where(gate, sc[S_SINSUM] / mc, f32(0.0))
    out_ref[4] = jnp.where(gate, sc[S_COSSUM] / mc, f32(0.0))
    out_ref[5] = f32(0.0)
    out_ref[6] = f32(0.0)
    out_ref[7] = f32(0.0)


def _final(cnt, sums, scalars):
    return pl.pallas_call(
        _final_body,
        in_specs=[
            pl.BlockSpec(memory_space=pltpu.VMEM),
            pl.BlockSpec(memory_space=pltpu.VMEM),
            pl.BlockSpec(memory_space=pltpu.SMEM),
        ],
        out_specs=pl.BlockSpec(memory_space=pltpu.SMEM),
        out_shape=jax.ShapeDtypeStruct((8,), jnp.float32),
    )(cnt, sums, scalars)


def kernel(input, tr_mask, tcl_mask, sin_map, cos_map, radii_map, train_mask):
    scalars, negv = _pass_a(
        input, tr_mask, tcl_mask, sin_map, cos_map, radii_map, train_mask)
    cnt, sums = _sc_hist(negv.reshape(-1))
    out = _final(cnt, sums, scalars)
    return (out[0], out[1], out[2], out[3], out[4])


# speedup vs baseline: 2.3849x; 1.0924x over previous
import jax
import jax.numpy as jnp
from jax import lax
from jax.experimental import pallas as pl
from jax.experimental.pallas import tpu as pltpu
from jax.experimental.pallas import tpu_sc as plsc

B, H, W = 8, 512, 512
N = B * H * W
RB = 64
NB = 32768
SHIFT = 16
NC, NS = 2, 16
NW = NC * NS
PER_W = N // NW
CH = 8192
UNROLL = 8

S_NPOS, S_LPOS, S_NAVAIL, S_TCLSUM, S_TCLTRN = 0, 1, 2, 3, 4
S_MCNT, S_RADSUM, S_SINSUM, S_COSSUM = 5, 6, 7, 8


def _ce2(a0, a1, tgt):
    m = jnp.maximum(a0, a1)
    lse = m + jnp.log1p(jnp.exp(-jnp.abs(a0 - a1)))
    picked = jnp.where(tgt > 0, a1, a0)
    return lse - picked


def _smooth_l1(d):
    ad = jnp.abs(d)
    return jnp.where(ad < 1.0, 0.5 * d * d, ad - 0.5)


def _pass_a_body(inp_ref, tr_ref, tcl_ref, sin_ref, cos_ref, rad_ref, trn_ref,
                 sc_ref, neg_ref):
    b = pl.program_id(0)
    h = pl.program_id(1)

    @pl.when(jnp.logical_and(b == 0, h == 0))
    def _init():
        for i in range(16):
            sc_ref[i] = jnp.float32(0.0)

    x = inp_ref[0]
    trm = tr_ref[0]
    tclm = tcl_ref[0]
    trnm = trn_ref[0]
    sinm = sin_ref[0]
    cosm = cos_ref[0]
    radm = rad_ref[0]

    f32 = jnp.float32
    train = trnm > 0
    pos = jnp.logical_and(trm > 0, train)
    neg = jnp.logical_and(trm == 0, train)

    l_tr = _ce2(x[0], x[1], trm)
    l_tcl = _ce2(x[2], x[3], tclm)

    neg_ref[...] = jnp.where(neg, l_tr, f32(-1.0)).reshape(RB * W)

    sc_ref[S_NPOS] = sc_ref[S_NPOS] + jnp.sum(pos.astype(f32))
    sc_ref[S_LPOS] = sc_ref[S_LPOS] + jnp.sum(jnp.where(pos, l_tr, 0.0))
    sc_ref[S_NAVAIL] = sc_ref[S_NAVAIL] + jnp.sum(neg.astype(f32))

    sc_ref[S_TCLSUM] = sc_ref[S_TCLSUM] + jnp.sum(jnp.where(pos, l_tcl, 0.0))
    sc_ref[S_TCLTRN] = sc_ref[S_TCLTRN] + jnp.sum(
        jnp.logical_and(tclm > 0, train).astype(f32))

    m = tclm > 0
    sc_ref[S_MCNT] = sc_ref[S_MCNT] + jnp.sum(m.astype(f32))

    scale = jnp.sqrt(1.0 / (x[4] * x[4] + x[5] * x[5]))
    sc_ref[S_RADSUM] = sc_ref[S_RADSUM] + jnp.sum(
        jnp.where(m, _smooth_l1(x[6] / radm - 1.0), 0.0))
    sc_ref[S_SINSUM] = sc_ref[S_SINSUM] + jnp.sum(
        jnp.where(m, _smooth_l1(x[4] * scale - sinm), 0.0))
    sc_ref[S_COSSUM] = sc_ref[S_COSSUM] + jnp.sum(
        jnp.where(m, _smooth_l1(x[5] * scale - cosm), 0.0))


def _pass_a(input, tr_mask, tcl_mask, sin_map, cos_map, radii_map, train_mask):
    hb = H // RB
    mask_spec = pl.BlockSpec((1, RB, W), lambda b, h: (b, h, 0))
    return pl.pallas_call(
        _pass_a_body,
        grid=(B, hb),
        in_specs=[
            pl.BlockSpec((1, 7, RB, W), lambda b, h: (b, 0, h, 0)),
            mask_spec, mask_spec, mask_spec, mask_spec, mask_spec, mask_spec,
        ],
        out_specs=[
            pl.BlockSpec(memory_space=pltpu.SMEM),
            pl.BlockSpec((RB * W,), lambda b, h: (b * hb + h,)),
        ],
        out_shape=[
            jax.ShapeDtypeStruct((16,), jnp.float32),
            jax.ShapeDtypeStruct((N,), jnp.float32),
        ],
    )(input, tr_mask, tcl_mask, sin_map, cos_map, radii_map, train_mask)


def _sc_hist_body(vals, cnt_out, sum_out, vbuf, hcnt, hsum):
    wid = lax.axis_index("s") * NC + lax.axis_index("c")

    zeros = jnp.zeros((16,), jnp.float32)
    ones = jnp.ones((16,), jnp.float32)

    @plsc.parallel_loop(0, NB // 16, unroll=8)
    def _zero(i):
        hcnt[pl.ds(i * 16, 16)] = zeros
        hsum[pl.ds(i * 16, 16)] = zeros

    base = wid * PER_W
    for c in range(PER_W // CH):
        pltpu.sync_copy(vals.at[pl.ds(base + c * CH, CH)], vbuf)

        @plsc.parallel_loop(0, CH // 16, unroll=UNROLL)
        def _elem(j):
            v = vbuf[pl.ds(j * 16, 16)]
            valid = v >= 0.0
            idx = lax.shift_right_arithmetic(
                lax.bitcast_convert_type(v, jnp.int32), SHIFT)
            plsc.addupdate_scatter(hcnt, [idx], ones, mask=valid)
            plsc.addupdate_scatter(hsum, [idx], v, mask=valid)

    pltpu.sync_copy(hcnt, cnt_out.at[wid])
    pltpu.sync_copy(hsum, sum_out.at[wid])


def _sc_hist(vals):
    return pl.kernel(
        _sc_hist_body,
        out_type=(
            jax.ShapeDtypeStruct((NW, NB), jnp.float32),
            jax.ShapeDtypeStruct((NW, NB), jnp.float32),
        ),
        mesh=plsc.VectorSubcoreMesh(core_axis_name="c", subcore_axis_name="s",
                                    num_cores=NC, num_subcores=NS),
        compiler_params=pltpu.CompilerParams(needs_layout_passes=False),
        scratch_types=[
            pltpu.VMEM((CH,), jnp.float32),
            pltpu.VMEM((NB,), jnp.float32),
            pltpu.VMEM((NB,), jnp.float32),
        ],
    )(vals)


def _suffix(table):
    rows = NB // 128
    v = table.reshape(rows, 128)
    r = jnp.arange(rows, dtype=jnp.int32)
    row_after = (r[None, :] > r[:, None]).astype(jnp.float32)
    c = jnp.arange(128, dtype=jnp.int32)
    within = (c[None, :] >= c[:, None]).astype(jnp.float32)
    out = jnp.dot(v, within.T, preferred_element_type=jnp.float32) + \
        jnp.dot(row_after, jnp.sum(v, axis=1))[:, None]
    return out.reshape(NB)


def _final_body(cnt_ref, sum_ref, sc_ref, out_ref):
    f32 = jnp.float32
    cnt = jnp.sum(cnt_ref[...], axis=0)
    sums = jnp.sum(sum_ref[...], axis=0)
    ccnt = _suffix(cnt)
    csum = _suffix(sums)

    sc = [sc_ref[i] for i in range(9)]
    npos = sc[S_NPOS]
    avail = sc[S_NAVAIL]
    k = jnp.where(npos > 0, jnp.minimum(avail, 3.0 * npos), f32(100.0))

    bf = jnp.sum((ccnt >= k).astype(f32)) - 1.0
    j = jnp.arange(NB, dtype=jnp.int32).astype(f32)
    above = j == (bf + 1.0)
    c_ab = jnp.sum(jnp.where(above, ccnt, 0.0))
    s_ab = jnp.sum(jnp.where(above, csum, 0.0))

    b_i = bf.astype(jnp.int32)
    v_lo = lax.bitcast_convert_type(
        lax.shift_left(b_i, SHIFT), jnp.float32)
    v_hi = lax.bitcast_convert_type(
        lax.shift_left(b_i + 1, SHIFT), jnp.float32)
    tau = 0.5 * (v_lo + v_hi)

    deficit = k - c_ab
    top = s_ab + deficit * tau
    top = jnp.where(k <= 0.0, 0.0,
                    jnp.where(k > avail, f32(-jnp.inf), top))

    loss_tr = (sc[S_LPOS] + top) / (npos + k)
    loss_tcl = jnp.where(
        npos > 0, sc[S_TCLSUM] / jnp.maximum(npos, 1.0), f32(0.0))
    gate = sc[S_TCLTRN] > 0
    mc = jnp.maximum(sc[S_MCNT], 1.0)
    out_ref[0] = loss_tr
    out_ref[1] = loss_tcl
    out_ref[2] = jnp.where(gate, sc[S_RADSUM] / mc, f32(0.0))
    out_ref[3] = jnp.where(gate, sc[S_SINSUM] / mc, f32(0.0))
    out_ref[4] = jnp.where(gate, sc[S_COSSUM] / mc, f32(0.0))
    out_ref[5] = f32(0.0)
    out_ref[6] = f32(0.0)
    out_ref[7] = f32(0.0)


def _final(cnt, sums, scalars):
    return pl.pallas_call(
        _final_body,
        in_specs=[
            pl.BlockSpec(memory_space=pltpu.VMEM),
            pl.BlockSpec(memory_space=pltpu.VMEM),
            pl.BlockSpec(memory_space=pltpu.SMEM),
        ],
        out_specs=pl.BlockSpec(memory_space=pltpu.SMEM),
        out_shape=jax.ShapeDtypeStruct((8,), jnp.float32),
    )(cnt, sums, scalars)


def kernel(input, tr_mask, tcl_mask, sin_map, cos_map, radii_map, train_mask):
    scalars, negv = _pass_a(
        input, tr_mask, tcl_mask, sin_map, cos_map, radii_map, train_mask)
    cnt, sums = _sc_hist(negv)
    out = _final(cnt, sums, scalars)
    return (out[0], out[1], out[2], out[3], out[4])


# speedup vs baseline: 2.7755x; 1.1638x over previous
import jax
import jax.numpy as jnp
from jax import lax
from jax.experimental import pallas as pl
from jax.experimental.pallas import tpu as pltpu
from jax.experimental.pallas import tpu_sc as plsc

B, H, W = 8, 512, 512
N = B * H * W
RB = 128
NB = 32768
SHIFT = 16
NC, NS = 2, 16
NW = NC * NS
PER_W = N // NW
CH = 8192
UNROLL = 8

S_NPOS, S_LPOS, S_NAVAIL, S_TCLSUM, S_TCLTRN = 0, 1, 2, 3, 4
S_MCNT, S_RADSUM, S_SINSUM, S_COSSUM = 5, 6, 7, 8


def _ce2(a0, a1, tgt):
    m = jnp.maximum(a0, a1)
    lse = m + jnp.log1p(jnp.exp(-jnp.abs(a0 - a1)))
    picked = jnp.where(tgt > 0, a1, a0)
    return lse - picked


def _smooth_l1(d):
    ad = jnp.abs(d)
    return jnp.where(ad < 1.0, 0.5 * d * d, ad - 0.5)


def _pass_a_body(inp_ref, tr_ref, tcl_ref, sin_ref, cos_ref, rad_ref, trn_ref,
                 sc_ref, neg_ref):
    b = pl.program_id(0)
    h = pl.program_id(1)

    @pl.when(jnp.logical_and(b == 0, h == 0))
    def _init():
        for i in range(16):
            sc_ref[i] = jnp.float32(0.0)

    x = inp_ref[0]
    trm = tr_ref[0]
    tclm = tcl_ref[0]
    trnm = trn_ref[0]
    sinm = sin_ref[0]
    cosm = cos_ref[0]
    radm = rad_ref[0]

    f32 = jnp.float32
    train = trnm > 0
    pos = jnp.logical_and(trm > 0, train)
    neg = jnp.logical_and(trm == 0, train)

    l_tr = _ce2(x[0], x[1], trm)
    l_tcl = _ce2(x[2], x[3], tclm)

    neg_ref[...] = jnp.where(neg, l_tr, f32(-1.0)).reshape(RB * W)

    sc_ref[S_NPOS] = sc_ref[S_NPOS] + jnp.sum(pos.astype(f32))
    sc_ref[S_LPOS] = sc_ref[S_LPOS] + jnp.sum(jnp.where(pos, l_tr, 0.0))
    sc_ref[S_NAVAIL] = sc_ref[S_NAVAIL] + jnp.sum(neg.astype(f32))

    sc_ref[S_TCLSUM] = sc_ref[S_TCLSUM] + jnp.sum(jnp.where(pos, l_tcl, 0.0))
    sc_ref[S_TCLTRN] = sc_ref[S_TCLTRN] + jnp.sum(
        jnp.logical_and(tclm > 0, train).astype(f32))

    m = tclm > 0
    sc_ref[S_MCNT] = sc_ref[S_MCNT] + jnp.sum(m.astype(f32))

    scale = jnp.sqrt(1.0 / (x[4] * x[4] + x[5] * x[5]))
    sc_ref[S_RADSUM] = sc_ref[S_RADSUM] + jnp.sum(
        jnp.where(m, _smooth_l1(x[6] / radm - 1.0), 0.0))
    sc_ref[S_SINSUM] = sc_ref[S_SINSUM] + jnp.sum(
        jnp.where(m, _smooth_l1(x[4] * scale - sinm), 0.0))
    sc_ref[S_COSSUM] = sc_ref[S_COSSUM] + jnp.sum(
        jnp.where(m, _smooth_l1(x[5] * scale - cosm), 0.0))


def _pass_a(input, tr_mask, tcl_mask, sin_map, cos_map, radii_map, train_mask):
    hb = H // RB
    mask_spec = pl.BlockSpec((1, RB, W), lambda b, h: (b, h, 0))
    return pl.pallas_call(
        _pass_a_body,
        grid=(B, hb),
        in_specs=[
            pl.BlockSpec((1, 7, RB, W), lambda b, h: (b, 0, h, 0)),
            mask_spec, mask_spec, mask_spec, mask_spec, mask_spec, mask_spec,
        ],
        out_specs=[
            pl.BlockSpec(memory_space=pltpu.SMEM),
            pl.BlockSpec((RB * W,), lambda b, h: (b * hb + h,)),
        ],
        out_shape=[
            jax.ShapeDtypeStruct((16,), jnp.float32),
            jax.ShapeDtypeStruct((N,), jnp.float32),
        ],
    )(input, tr_mask, tcl_mask, sin_map, cos_map, radii_map, train_mask)


def _sc_hist_body(vals, cnt_out, sum_out, vbuf, hcnt, hsum):
    wid = lax.axis_index("s") * NC + lax.axis_index("c")

    zeros = jnp.zeros((16,), jnp.float32)
    ones = jnp.ones((16,), jnp.float32)

    @plsc.parallel_loop(0, NB // 16, unroll=8)
    def _zero(i):
        hcnt[pl.ds(i * 16, 16)] = zeros
        hsum[pl.ds(i * 16, 16)] = zeros

    base = wid * PER_W
    for c in range(PER_W // CH):
        pltpu.sync_copy(vals.at[pl.ds(base + c * CH, CH)], vbuf)

        @plsc.parallel_loop(0, CH // 16, unroll=UNROLL)
        def _elem(j):
            v = vbuf[pl.ds(j * 16, 16)]
            valid = v >= 0.0
            idx = lax.shift_right_arithmetic(
                lax.bitcast_convert_type(v, jnp.int32), SHIFT)
            plsc.addupdate_scatter(hcnt, [idx], ones, mask=valid)
            plsc.addupdate_scatter(hsum, [idx], v, mask=valid)

    pltpu.sync_copy(hcnt, cnt_out.at[wid])
    pltpu.sync_copy(hsum, sum_out.at[wid])


def _sc_hist(vals):
    return pl.kernel(
        _sc_hist_body,
        out_type=(
            jax.ShapeDtypeStruct((NW, NB), jnp.float32),
            jax.ShapeDtypeStruct((NW, NB), jnp.float32),
        ),
        mesh=plsc.VectorSubcoreMesh(core_axis_name="c", subcore_axis_name="s",
                                    num_cores=NC, num_subcores=NS),
        compiler_params=pltpu.CompilerParams(needs_layout_passes=False),
        scratch_types=[
            pltpu.VMEM((CH,), jnp.float32),
            pltpu.VMEM((NB,), jnp.float32),
            pltpu.VMEM((NB,), jnp.float32),
        ],
    )(vals)


def _suffix(table):
    rows = NB // 128
    v = table.reshape(rows, 128)
    r = jnp.arange(rows, dtype=jnp.int32)
    row_after = (r[None, :] > r[:, None]).astype(jnp.float32)
    c = jnp.arange(128, dtype=jnp.int32)
    within = (c[None, :] >= c[:, None]).astype(jnp.float32)
    out = jnp.dot(v, within.T, preferred_element_type=jnp.float32) + \
        jnp.dot(row_after, jnp.sum(v, axis=1))[:, None]
    return out.reshape(NB)


def _final_body(cnt_ref, sum_ref, sc_ref, out_ref):
    f32 = jnp.float32
    cnt = jnp.sum(cnt_ref[...], axis=0)
    sums = jnp.sum(sum_ref[...], axis=0)
    ccnt = _suffix(cnt)
    csum = _suffix(sums)

    sc = [sc_ref[i] for i in range(9)]
    npos = sc[S_NPOS]
    avail = sc[S_NAVAIL]
    k = jnp.where(npos > 0, jnp.minimum(avail, 3.0 * npos), f32(100.0))

    bf = jnp.sum((ccnt >= k).astype(f32)) - 1.0
    j = jnp.arange(NB, dtype=jnp.int32).astype(f32)
    above = j == (bf + 1.0)
    c_ab = jnp.sum(jnp.where(above, ccnt, 0.0))
    s_ab = jnp.sum(jnp.where(above, csum, 0.0))

    b_i = bf.astype(jnp.int32)
    v_lo = lax.bitcast_convert_type(
        lax.shift_left(b_i, SHIFT), jnp.float32)
    v_hi = lax.bitcast_convert_type(
        lax.shift_left(b_i + 1, SHIFT), jnp.float32)
    tau = 0.5 * (v_lo + v_hi)

    deficit = k - c_ab
    top = s_ab + deficit * tau
    top = jnp.where(k <= 0.0, 0.0,
                    jnp.where(k > avail, f32(-jnp.inf), top))

    loss_tr = (sc[S_LPOS] + top) / (npos + k)
    loss_tcl = jnp.where(
        npos > 0, sc[S_TCLSUM] / jnp.maximum(npos, 1.0), f32(0.0))
    gate = sc[S_TCLTRN] > 0
    mc = jnp.maximum(sc[S_MCNT], 1.0)
    out_ref[0] = loss_tr
    out_ref[1] = loss_tcl
    out_ref[2] = jnp.where(gate, sc[S_RADSUM] / mc, f32(0.0))
    out_ref[3] = jnp.where(gate, sc[S_SINSUM] / mc, f32(0.0))
    out_ref[4] = jnp.where(gate, sc[S_COSSUM] / mc, f32(0.0))
    out_ref[5] = f32(0.0)
    out_ref[6] = f32(0.0)
    out_ref[7] = f32(0.0)


def _final(cnt, sums, scalars):
    return pl.pallas_call(
        _final_body,
        in_specs=[
            pl.BlockSpec(memory_space=pltpu.VMEM),
            pl.BlockSpec(memory_space=pltpu.VMEM),
            pl.BlockSpec(memory_space=pltpu.SMEM),
        ],
        out_specs=pl.BlockSpec(memory_space=pltpu.SMEM),
        out_shape=jax.ShapeDtypeStruct((8,), jnp.float32),
    )(cnt, sums, scalars)


def kernel(input, tr_mask, tcl_mask, sin_map, cos_map, radii_map, train_mask):
    scalars, negv = _pass_a(
        input, tr_mask, tcl_mask, sin_map, cos_map, radii_map, train_mask)
    cnt, sums = _sc_hist(negv)
    out = _final(cnt, sums, scalars)
    return (out[0], out[1], out[2], out[3], out[4])


# speedup vs baseline: 2.9913x; 1.0778x over previous
import jax
import jax.numpy as jnp
from jax import lax
from jax.experimental import pallas as pl
from jax.experimental.pallas import tpu as pltpu
from jax.experimental.pallas import tpu_sc as plsc

B, H, W = 8, 512, 512
N = B * H * W
RB = 256
NB = 32768
SHIFT = 16
NC, NS = 2, 16
NW = NC * NS
PER_W = N // NW
CH = 8192
UNROLL = 8

S_NPOS, S_LPOS, S_NAVAIL, S_TCLSUM, S_TCLTRN = 0, 1, 2, 3, 4
S_MCNT, S_RADSUM, S_SINSUM, S_COSSUM = 5, 6, 7, 8


def _ce2(a0, a1, tgt):
    m = jnp.maximum(a0, a1)
    lse = m + jnp.log1p(jnp.exp(-jnp.abs(a0 - a1)))
    picked = jnp.where(tgt > 0, a1, a0)
    return lse - picked


def _smooth_l1(d):
    ad = jnp.abs(d)
    return jnp.where(ad < 1.0, 0.5 * d * d, ad - 0.5)


def _pass_a_body(inp_ref, tr_ref, tcl_ref, sin_ref, cos_ref, rad_ref, trn_ref,
                 sc_ref, neg_ref):
    b = pl.program_id(0)
    h = pl.program_id(1)

    @pl.when(jnp.logical_and(b == 0, h == 0))
    def _init():
        for i in range(16):
            sc_ref[i] = jnp.float32(0.0)

    x = inp_ref[0]
    trm = tr_ref[0]
    tclm = tcl_ref[0]
    trnm = trn_ref[0]
    sinm = sin_ref[0]
    cosm = cos_ref[0]
    radm = rad_ref[0]

    f32 = jnp.float32
    train = trnm > 0
    pos = jnp.logical_and(trm > 0, train)
    neg = jnp.logical_and(trm == 0, train)

    l_tr = _ce2(x[0], x[1], trm)
    l_tcl = _ce2(x[2], x[3], tclm)

    neg_ref[...] = jnp.where(neg, l_tr, f32(-1.0)).reshape(RB * W)

    sc_ref[S_NPOS] = sc_ref[S_NPOS] + jnp.sum(pos.astype(f32))
    sc_ref[S_LPOS] = sc_ref[S_LPOS] + jnp.sum(jnp.where(pos, l_tr, 0.0))
    sc_ref[S_NAVAIL] = sc_ref[S_NAVAIL] + jnp.sum(neg.astype(f32))

    sc_ref[S_TCLSUM] = sc_ref[S_TCLSUM] + jnp.sum(jnp.where(pos, l_tcl, 0.0))
    sc_ref[S_TCLTRN] = sc_ref[S_TCLTRN] + jnp.sum(
        jnp.logical_and(tclm > 0, train).astype(f32))

    m = tclm > 0
    sc_ref[S_MCNT] = sc_ref[S_MCNT] + jnp.sum(m.astype(f32))

    scale = jnp.sqrt(1.0 / (x[4] * x[4] + x[5] * x[5]))
    sc_ref[S_RADSUM] = sc_ref[S_RADSUM] + jnp.sum(
        jnp.where(m, _smooth_l1(x[6] / radm - 1.0), 0.0))
    sc_ref[S_SINSUM] = sc_ref[S_SINSUM] + jnp.sum(
        jnp.where(m, _smooth_l1(x[4] * scale - sinm), 0.0))
    sc_ref[S_COSSUM] = sc_ref[S_COSSUM] + jnp.sum(
        jnp.where(m, _smooth_l1(x[5] * scale - cosm), 0.0))


def _pass_a(input, tr_mask, tcl_mask, sin_map, cos_map, radii_map, train_mask):
    hb = H // RB
    mask_spec = pl.BlockSpec((1, RB, W), lambda b, h: (b, h, 0))
    return pl.pallas_call(
        _pass_a_body,
        grid=(B, hb),
        in_specs=[
            pl.BlockSpec((1, 7, RB, W), lambda b, h: (b, 0, h, 0)),
            mask_spec, mask_spec, mask_spec, mask_spec, mask_spec, mask_spec,
        ],
        out_specs=[
            pl.BlockSpec(memory_space=pltpu.SMEM),
            pl.BlockSpec((RB * W,), lambda b, h: (b * hb + h,)),
        ],
        out_shape=[
            jax.ShapeDtypeStruct((16,), jnp.float32),
            jax.ShapeDtypeStruct((N,), jnp.float32),
        ],
    )(input, tr_mask, tcl_mask, sin_map, cos_map, radii_map, train_mask)


def _sc_hist_body(vals, cnt_out, sum_out, vbuf, hcnt, hsum):
    wid = lax.axis_index("s") * NC + lax.axis_index("c")

    zeros = jnp.zeros((16,), jnp.float32)
    ones = jnp.ones((16,), jnp.float32)

    @plsc.parallel_loop(0, NB // 16, unroll=8)
    def _zero(i):
        hcnt[pl.ds(i * 16, 16)] = zeros
        hsum[pl.ds(i * 16, 16)] = zeros

    base = wid * PER_W
    for c in range(PER_W // CH):
        pltpu.sync_copy(vals.at[pl.ds(base + c * CH, CH)], vbuf)

        @plsc.parallel_loop(0, CH // 16, unroll=UNROLL)
        def _elem(j):
            v = vbuf[pl.ds(j * 16, 16)]
            valid = v >= 0.0
            idx = lax.shift_right_arithmetic(
                lax.bitcast_convert_type(v, jnp.int32), SHIFT)
            plsc.addupdate_scatter(hcnt, [idx], ones, mask=valid)
            plsc.addupdate_scatter(hsum, [idx], v, mask=valid)

    pltpu.sync_copy(hcnt, cnt_out.at[wid])
    pltpu.sync_copy(hsum, sum_out.at[wid])


def _sc_hist(vals):
    return pl.kernel(
        _sc_hist_body,
        out_type=(
            jax.ShapeDtypeStruct((NW, NB), jnp.float32),
            jax.ShapeDtypeStruct((NW, NB), jnp.float32),
        ),
        mesh=plsc.VectorSubcoreMesh(core_axis_name="c", subcore_axis_name="s",
                                    num_cores=NC, num_subcores=NS),
        compiler_params=pltpu.CompilerParams(needs_layout_passes=False),
        scratch_types=[
            pltpu.VMEM((CH,), jnp.float32),
            pltpu.VMEM((NB,), jnp.float32),
            pltpu.VMEM((NB,), jnp.float32),
        ],
    )(vals)


def _suffix(table):
    rows = NB // 128
    v = table.reshape(rows, 128)
    r = jnp.arange(rows, dtype=jnp.int32)
    row_after = (r[None, :] > r[:, None]).astype(jnp.float32)
    c = jnp.arange(128, dtype=jnp.int32)
    within = (c[None, :] >= c[:, None]).astype(jnp.float32)
    out = jnp.dot(v, within.T, preferred_element_type=jnp.float32) + \
        jnp.dot(row_after, jnp.sum(v, axis=1))[:, None]
    return out.reshape(NB)


def _final_body(cnt_ref, sum_ref, sc_ref, out_ref):
    f32 = jnp.float32
    cnt = jnp.sum(cnt_ref[...], axis=0)
    sums = jnp.sum(sum_ref[...], axis=0)
    ccnt = _suffix(cnt)
    csum = _suffix(sums)

    sc = [sc_ref[i] for i in range(9)]
    npos = sc[S_NPOS]
    avail = sc[S_NAVAIL]
    k = jnp.where(npos > 0, jnp.minimum(avail, 3.0 * npos), f32(100.0))

    bf = jnp.sum((ccnt >= k).astype(f32)) - 1.0
    j = jnp.arange(NB, dtype=jnp.int32).astype(f32)
    above = j == (bf + 1.0)
    c_ab = jnp.sum(jnp.where(above, ccnt, 0.0))
    s_ab = jnp.sum(jnp.where(above, csum, 0.0))

    b_i = bf.astype(jnp.int32)
    v_lo = lax.bitcast_convert_type(
        lax.shift_left(b_i, SHIFT), jnp.float32)
    v_hi = lax.bitcast_convert_type(
        lax.shift_left(b_i + 1, SHIFT), jnp.float32)
    tau = 0.5 * (v_lo + v_hi)

    deficit = k - c_ab
    top = s_ab + deficit * tau
    top = jnp.where(k <= 0.0, 0.0,
                    jnp.where(k > avail, f32(-jnp.inf), top))

    loss_tr = (sc[S_LPOS] + top) / (npos + k)
    loss_tcl = jnp.where(
        npos > 0, sc[S_TCLSUM] / jnp.maximum(npos, 1.0), f32(0.0))
    gate = sc[S_TCLTRN] > 0
    mc = jnp.maximum(sc[S_MCNT], 1.0)
    out_ref[0] = loss_tr
    out_ref[1] = loss_tcl
    out_ref[2] = jnp.where(gate, sc[S_RADSUM] / mc, f32(0.0))
    out_ref[3] = jnp.where(gate, sc[S_SINSUM] / mc, f32(0.0))
    out_ref[4] = jnp.where(gate, sc[S_COSSUM] / mc, f32(0.0))
    out_ref[5] = f32(0.0)
    out_ref[6] = f32(0.0)
    out_ref[7] = f32(0.0)


def _final(cnt, sums, scalars):
    return pl.pallas_call(
        _final_body,
        in_specs=[
            pl.BlockSpec(memory_space=pltpu.VMEM),
            pl.BlockSpec(memory_space=pltpu.VMEM),
            pl.BlockSpec(memory_space=pltpu.SMEM),
        ],
        out_specs=pl.BlockSpec(memory_space=pltpu.SMEM),
        out_shape=jax.ShapeDtypeStruct((8,), jnp.float32),
    )(cnt, sums, scalars)


def kernel(input, tr_mask, tcl_mask, sin_map, cos_map, radii_map, train_mask):
    scalars, negv = _pass_a(
        input, tr_mask, tcl_mask, sin_map, cos_map, radii_map, train_mask)
    cnt, sums = _sc_hist(negv)
    out = _final(cnt, sums, scalars)
    return (out[0], out[1], out[2], out[3], out[4])


# speedup vs baseline: 3.2452x; 1.0849x over previous
import jax
import jax.numpy as jnp
from jax import lax
from jax.experimental import pallas as pl
from jax.experimental.pallas import tpu as pltpu
from jax.experimental.pallas import tpu_sc as plsc

B, H, W = 8, 512, 512
N = B * H * W
RB = 256
NB = 32768
SHIFT = 16
NC, NS = 2, 16
NW = NC * NS
PER_W = N // NW
CH = 8192
UNROLL = 8

S_NPOS, S_LPOS, S_NAVAIL = 0, 1, 2
S_TCLSUM, S_TCLTRN, S_MCNT, S_RADSUM, S_SINSUM, S_COSSUM = 0, 1, 2, 3, 4, 5


def _ce2(a0, a1, tgt):
    m = jnp.maximum(a0, a1)
    lse = m + jnp.log1p(jnp.exp(-jnp.abs(a0 - a1)))
    picked = jnp.where(tgt > 0, a1, a0)
    return lse - picked


def _smooth_l1(d):
    ad = jnp.abs(d)
    return jnp.where(ad < 1.0, 0.5 * d * d, ad - 0.5)


def _a1_body(x01_ref, tr_ref, trn_ref, sc_ref, neg_ref):
    b = pl.program_id(0)
    h = pl.program_id(1)

    @pl.when(jnp.logical_and(b == 0, h == 0))
    def _init():
        for i in range(16):
            sc_ref[i] = jnp.float32(0.0)

    x = x01_ref[0]
    trm = tr_ref[0]
    trnm = trn_ref[0]
    f32 = jnp.float32
    train = trnm > 0
    pos = jnp.logical_and(trm > 0, train)
    neg = jnp.logical_and(trm == 0, train)
    l_tr = _ce2(x[0], x[1], trm)
    neg_ref[...] = jnp.where(neg, l_tr, f32(-1.0)).reshape(RB * W)
    sc_ref[S_NPOS] = sc_ref[S_NPOS] + jnp.sum(pos.astype(f32))
    sc_ref[S_LPOS] = sc_ref[S_LPOS] + jnp.sum(jnp.where(pos, l_tr, 0.0))
    sc_ref[S_NAVAIL] = sc_ref[S_NAVAIL] + jnp.sum(neg.astype(f32))


def _a1(input, tr_mask, train_mask):
    hb = H // RB
    mask_spec = pl.BlockSpec((1, RB, W), lambda b, h: (b, h, 0))
    return pl.pallas_call(
        _a1_body,
        grid=(B, hb),
        in_specs=[
            pl.BlockSpec((1, 2, RB, W), lambda b, h: (b, 0, h, 0)),
            mask_spec, mask_spec,
        ],
        out_specs=[
            pl.BlockSpec(memory_space=pltpu.SMEM),
            pl.BlockSpec((RB * W,), lambda b, h: (b * hb + h,)),
        ],
        out_shape=[
            jax.ShapeDtypeStruct((16,), jnp.float32),
            jax.ShapeDtypeStruct((N,), jnp.float32),
        ],
    )(input, tr_mask, train_mask)


def _a2_body(x23_ref, x45_ref, x6_ref, tcl_ref, tr_ref, trn_ref,
             sin_ref, cos_ref, rad_ref, sc_ref):
    b = pl.program_id(0)
    h = pl.program_id(1)

    @pl.when(jnp.logical_and(b == 0, h == 0))
    def _init():
        for i in range(16):
            sc_ref[i] = jnp.float32(0.0)

    x23 = x23_ref[0]
    x45 = x45_ref[0]
    x6 = x6_ref[0, 0]
    trm = tr_ref[0]
    tclm = tcl_ref[0]
    trnm = trn_ref[0]
    sinm = sin_ref[0]
    cosm = cos_ref[0]
    radm = rad_ref[0]

    f32 = jnp.float32
    train = trnm > 0
    pos = jnp.logical_and(trm > 0, train)
    l_tcl = _ce2(x23[0], x23[1], tclm)
    sc_ref[S_TCLSUM] = sc_ref[S_TCLSUM] + jnp.sum(jnp.where(pos, l_tcl, 0.0))
    sc_ref[S_TCLTRN] = sc_ref[S_TCLTRN] + jnp.sum(
        jnp.logical_and(tclm > 0, train).astype(f32))

    m = tclm > 0
    sc_ref[S_MCNT] = sc_ref[S_MCNT] + jnp.sum(m.astype(f32))
    scale = jnp.sqrt(1.0 / (x45[0] * x45[0] + x45[1] * x45[1]))
    sc_ref[S_RADSUM] = sc_ref[S_RADSUM] + jnp.sum(
        jnp.where(m, _smooth_l1(x6 / radm - 1.0), 0.0))
    sc_ref[S_SINSUM] = sc_ref[S_SINSUM] + jnp.sum(
        jnp.where(m, _smooth_l1(x45[0] * scale - sinm), 0.0))
    sc_ref[S_COSSUM] = sc_ref[S_COSSUM] + jnp.sum(
        jnp.where(m, _smooth_l1(x45[1] * scale - cosm), 0.0))


def _a2(input, tcl_mask, tr_mask, train_mask, sin_map, cos_map, radii_map):
    mask_spec = pl.BlockSpec((1, RB, W), lambda b, h: (b, h, 0))
    return pl.pallas_call(
        _a2_body,
        grid=(B, H // RB),
        in_specs=[
            pl.BlockSpec((1, 2, RB, W), lambda b, h: (b, 1, h, 0)),
            pl.BlockSpec((1, 2, RB, W), lambda b, h: (b, 2, h, 0)),
            pl.BlockSpec((1, 1, RB, W), lambda b, h: (b, 6, h, 0)),
            mask_spec, mask_spec, mask_spec, mask_spec, mask_spec, mask_spec,
        ],
        out_specs=pl.BlockSpec(memory_space=pltpu.SMEM),
        out_shape=jax.ShapeDtypeStruct((16,), jnp.float32),
    )(input, input, input, tcl_mask, tr_mask, train_mask,
      sin_map, cos_map, radii_map)


def _sc_hist_body(vals, cnt_out, sum_out, vbuf, hcnt, hsum):
    wid = lax.axis_index("s") * NC + lax.axis_index("c")

    zeros = jnp.zeros((16,), jnp.float32)
    ones = jnp.ones((16,), jnp.float32)

    @plsc.parallel_loop(0, NB // 16, unroll=8)
    def _zero(i):
        hcnt[pl.ds(i * 16, 16)] = zeros
        hsum[pl.ds(i * 16, 16)] = zeros

    base = wid * PER_W
    for c in range(PER_W // CH):
        pltpu.sync_copy(vals.at[pl.ds(base + c * CH, CH)], vbuf)

        @plsc.parallel_loop(0, CH // 16, unroll=UNROLL)
        def _elem(j):
            v = vbuf[pl.ds(j * 16, 16)]
            valid = v >= 0.0
            idx = lax.shift_right_arithmetic(
                lax.bitcast_convert_type(v, jnp.int32), SHIFT)
            plsc.addupdate_scatter(hcnt, [idx], ones, mask=valid)
            plsc.addupdate_scatter(hsum, [idx], v, mask=valid)

    pltpu.sync_copy(hcnt, cnt_out.at[wid])
    pltpu.sync_copy(hsum, sum_out.at[wid])


def _sc_hist(vals):
    return pl.kernel(
        _sc_hist_body,
        out_type=(
            jax.ShapeDtypeStruct((NW, NB), jnp.float32),
            jax.ShapeDtypeStruct((NW, NB), jnp.float32),
        ),
        mesh=plsc.VectorSubcoreMesh(core_axis_name="c", subcore_axis_name="s",
                                    num_cores=NC, num_subcores=NS),
        compiler_params=pltpu.CompilerParams(needs_layout_passes=False),
        scratch_types=[
            pltpu.VMEM((CH,), jnp.float32),
            pltpu.VMEM((NB,), jnp.float32),
            pltpu.VMEM((NB,), jnp.float32),
        ],
    )(vals)


def _suffix(table):
    rows = NB // 128
    v = table.reshape(rows, 128)
    r = jnp.arange(rows, dtype=jnp.int32)
    row_after = (r[None, :] > r[:, None]).astype(jnp.float32)
    c = jnp.arange(128, dtype=jnp.int32)
    within = (c[None, :] >= c[:, None]).astype(jnp.float32)
    out = jnp.dot(v, within.T, preferred_element_type=jnp.float32) + \
        jnp.dot(row_after, jnp.sum(v, axis=1))[:, None]
    return out.reshape(NB)


def _final_body(cnt_ref, sum_ref, s1_ref, s2_ref, out_ref):
    f32 = jnp.float32
    cnt = jnp.sum(cnt_ref[...], axis=0)
    sums = jnp.sum(sum_ref[...], axis=0)
    ccnt = _suffix(cnt)
    csum = _suffix(sums)

    npos = s1_ref[S_NPOS]
    avail = s1_ref[S_NAVAIL]
    k = jnp.where(npos > 0, jnp.minimum(avail, 3.0 * npos), f32(100.0))

    bf = jnp.sum((ccnt >= k).astype(f32)) - 1.0
    j = jnp.arange(NB, dtype=jnp.int32).astype(f32)
    above = j == (bf + 1.0)
    c_ab = jnp.sum(jnp.where(above, ccnt, 0.0))
    s_ab = jnp.sum(jnp.where(above, csum, 0.0))

    b_i = bf.astype(jnp.int32)
    v_lo = lax.bitcast_convert_type(lax.shift_left(b_i, SHIFT), jnp.float32)
    v_hi = lax.bitcast_convert_type(lax.shift_left(b_i + 1, SHIFT), jnp.float32)
    tau = 0.5 * (v_lo + v_hi)

    deficit = k - c_ab
    top = s_ab + deficit * tau
    top = jnp.where(k <= 0.0, 0.0,
                    jnp.where(k > avail, f32(-jnp.inf), top))

    loss_tr = (s1_ref[S_LPOS] + top) / (npos + k)
    loss_tcl = jnp.where(
        npos > 0, s2_ref[S_TCLSUM] / jnp.maximum(npos, 1.0), f32(0.0))
    gate = s2_ref[S_TCLTRN] > 0
    mc = jnp.maximum(s2_ref[S_MCNT], 1.0)
    out_ref[0] = loss_tr
    out_ref[1] = loss_tcl
    out_ref[2] = jnp.where(gate, s2_ref[S_RADSUM] / mc, f32(0.0))
    out_ref[3] = jnp.where(gate, s2_ref[S_SINSUM] / mc, f32(0.0))
    out_ref[4] = jnp.where(gate, s2_ref[S_COSSUM] / mc, f32(0.0))
    out_ref[5] = f32(0.0)
    out_ref[6] = f32(0.0)
    out_ref[7] = f32(0.0)


def _final(cnt, sums, s1, s2):
    return pl.pallas_call(
        _final_body,
        in_specs=[
            pl.BlockSpec(memory_space=pltpu.VMEM),
            pl.BlockSpec(memory_space=pltpu.VMEM),
            pl.BlockSpec(memory_space=pltpu.SMEM),
            pl.BlockSpec(memory_space=pltpu.SMEM),
        ],
        out_specs=pl.BlockSpec(memory_space=pltpu.SMEM),
        out_shape=jax.ShapeDtypeStruct((8,), jnp.float32),
    )(cnt, sums, s1, s2)


def kernel(input, tr_mask, tcl_mask, sin_map, cos_map, radii_map, train_mask):
    s1, negv = _a1(input, tr_mask, train_mask)
    cnt, sums = _sc_hist(negv)
    s2 = _a2(input, tcl_mask, tr_mask, train_mask, sin_map, cos_map, radii_map)
    out = _final(cnt, sums, s1, s2)
    return (out[0], out[1], out[2], out[3], out[4])
